# baseline (speedup 1.0000x reference)
import functools

import jax
import jax.numpy as jnp
from jax import lax
from jax.experimental import pallas as pl
from jax.experimental.pallas import tpu as pltpu
from jax.experimental.pallas import tpu_sc as plsc

B_ROWS = 16384
B_COLS = 200
D = 24
NKEY = 24
PSTRIDE = 48
B = B_ROWS * B_COLS
NC, NS = 2, 16
NW = NC * NS
ROWS_PER_STEP = 2048
PAIR_GROUPS = ROWS_PER_STEP // 32
OUT_PER_STEP = ROWS_PER_STEP * D
B_PER_W = B // NW
STEPS = B_PER_W // ROWS_PER_STEP


def _sc_lookup(idx_flat, ptab):
    mesh = plsc.VectorSubcoreMesh(core_axis_name="c", subcore_axis_name="s")

    @functools.partial(
        pl.kernel,
        mesh=mesh,
        compiler_params=pltpu.CompilerParams(
            use_tc_tiling_on_sc=False, needs_layout_passes=False),
        out_type=jax.ShapeDtypeStruct((B * D,), jnp.float32),
        scratch_types=[
            pltpu.VMEM((NKEY * NKEY * PSTRIDE,), jnp.float32),
            pltpu.VMEM((ROWS_PER_STEP,), jnp.int32),
            pltpu.VMEM((ROWS_PER_STEP,), jnp.int32),
            pltpu.VMEM((OUT_PER_STEP,), jnp.float32),
            pltpu.VMEM((OUT_PER_STEP,), jnp.float32),
            pltpu.SemaphoreType.DMA,
            pltpu.SemaphoreType.DMA,
            pltpu.SemaphoreType.DMA,
            pltpu.SemaphoreType.DMA,
        ],
    )
    def k(idx_hbm, ptab_hbm, out_hbm, tab_v,
          idx_v0, idx_v1, out_v0, out_v1, si0, si1, so0, so1):
        wid = lax.axis_index("s") * NC + lax.axis_index("c")
        row0 = wid * B_PER_W
        pltpu.sync_copy(ptab_hbm, tab_v)

        iota = lax.iota(jnp.int32, 16)
        perm_e = (iota * 2) & 15
        perm_o = perm_e + 1
        lo_half = iota < 8

        _dnums = lax.GatherDimensionNumbers(
            offset_dims=(), collapsed_slice_dims=(0,), start_index_map=(0,))

        def _vperm(vec, perm):
            return lax.gather(
                vec, perm[:, None], _dnums, (1,),
                mode=lax.GatherScatterMode.PROMISE_IN_BOUNDS)

        idx_bufs = (idx_v0, idx_v1)
        out_bufs = (out_v0, out_v1)
        si = (si0, si1)
        so = (so0, so1)

        def idx_slice(it):
            base = pl.multiple_of(row0 + it * ROWS_PER_STEP, 8)
            return idx_hbm.at[pl.ds(base, ROWS_PER_STEP)]

        def out_slice(it):
            base = pl.multiple_of((row0 + it * ROWS_PER_STEP) * D, 8)
            return out_hbm.at[pl.ds(base, OUT_PER_STEP)]

        pltpu.async_copy(idx_slice(0), idx_v0, si0)
        pltpu.async_copy(idx_slice(1), idx_v1, si1)

        def outer(i, carry):
            for b in range(2):
                it = 2 * i + b
                ib, ob, sib, sob = idx_bufs[b], out_bufs[b], si[b], so[b]
                pltpu.make_async_copy(idx_slice(it), ib, sib).wait()

                @pl.when(i > 0)
                def _wait_out():
                    pltpu.make_async_copy(ob, out_slice(it - 2), sob).wait()

                def group(g, c):
                    va = ib[pl.ds(g * 32, 16)]
                    vb = ib[pl.ds(g * 32 + 16, 16)]
                    ga_e = _vperm(va, perm_e)
                    gb_e = _vperm(vb, perm_e)
                    ga_o = _vperm(va, perm_o)
                    gb_o = _vperm(vb, perm_o)
                    i0 = jnp.where(lo_half, ga_e, gb_e)
                    i1 = jnp.where(lo_half, ga_o, gb_o)
                    addrs = i0 * (NKEY * PSTRIDE) + i1 * PSTRIDE
                    obase = g * (16 * 2 * D)
                    for u in range(16):
                        a = addrs[u]
                        o = obase + u * (2 * D)
                        ob[pl.ds(o, 16)] = tab_v[pl.ds(a, 16)]
                        ob[pl.ds(o + 16, 16)] = tab_v[pl.ds(a + 16, 16)]
                        ob[pl.ds(o + 32, 16)] = tab_v[pl.ds(a + 32, 16)]
                    return c

                lax.fori_loop(0, PAIR_GROUPS, group, 0)
                pltpu.async_copy(ob, out_slice(it), sob)

                @pl.when(it + 2 < STEPS)
                def _next_idx():
                    pltpu.async_copy(idx_slice(it + 2), ib, sib)
            return carry

        lax.fori_loop(0, STEPS // 2, outer, 0)
        pltpu.make_async_copy(out_v0, out_slice(STEPS - 2), so0).wait()
        pltpu.make_async_copy(out_v1, out_slice(STEPS - 1), so1).wait()

    return k(idx_flat, ptab)


BLK_Q = 2048
BLK_OUT = BLK_Q * D
NBLK = B // BLK_Q


def _tc_expand(idx2d, bcat, acat, vpat, t3, e0, e1, e2, nblk, blk0):
    f32 = jnp.float32

    def body(idx_ref, bcat_ref, acat_ref, vpat_ref, t3_ref,
             e0_ref, e1_ref, e2_ref, o_ref):
        idxn = idx_ref[...].astype(jnp.bfloat16)
        y = jnp.dot(idxn, bcat_ref[...],
                    preferred_element_type=f32).astype(jnp.bfloat16)
        ac = acat_ref[...]
        idxe = jnp.zeros((128, 512), f32)
        for m in range(8):
            idxe += jnp.dot(ac[:, 16 * m:16 * m + 16],
                            y[:, 512 * m:512 * m + 512],
                            preferred_element_type=f32)
        onehot = jnp.where(idxe == vpat_ref[...], 1.0, 0.0).astype(
            jnp.bfloat16)
        g3 = jnp.dot(onehot, t3_ref[...], preferred_element_type=f32)
        g3b = g3.astype(jnp.bfloat16)
        outp = (jnp.dot(e0_ref[...], g3b[:, 0:128], preferred_element_type=f32)
                + jnp.dot(e1_ref[...], g3b[:, 128:256],
                          preferred_element_type=f32)
                + jnp.dot(e2_ref[...], g3b[:, 256:384],
                          preferred_element_type=f32))
        o_ref[...] = outp.reshape(BLK_OUT)

    return pl.pallas_call(
        body,
        grid=(nblk,),
        in_specs=[
            pl.BlockSpec((16, 128), lambda i: (i + blk0, 0)),
            pl.BlockSpec((128, 4096), lambda i: (0, 0)),
            pl.BlockSpec((128, 128), lambda i: (0, 0)),
            pl.BlockSpec((1, 512), lambda i: (0, 0)),
            pl.BlockSpec((512, 384), lambda i: (0, 0)),
            pl.BlockSpec((384, 128), lambda i: (0, 0)),
            pl.BlockSpec((384, 128), lambda i: (0, 0)),
            pl.BlockSpec((384, 128), lambda i: (0, 0)),
        ],
        out_specs=pl.BlockSpec((BLK_OUT,), lambda i: (i + blk0,)),
        out_shape=jax.ShapeDtypeStruct((B * D,), jnp.float32),
    )(idx2d, bcat, acat, vpat, t3, e0, e1, e2)


def _tc_consts(table):
    import numpy as np
    t3j = jnp.zeros((512, 384), jnp.float32)
    for j in range(16):
        t3j = t3j.at[32 * j:32 * j + NKEY, 24 * j:24 * j + NKEY].set(
            table[:NKEY, :NKEY])
    c = np.arange(512)
    l = np.arange(128)
    bcat = np.concatenate(
        [(l[:, None] == 16 * m + c[None, :] // 32).astype(np.float32)
         for m in range(8)], axis=1)
    s = np.arange(128)
    p = np.arange(16)
    acat = np.concatenate(
        [((s[:, None] % 8 == m) & (p[None, :] == s[:, None] // 8)
          ).astype(np.float32) for m in range(8)], axis=1)
    vpat = (c % 32).astype(np.float32)[None, :]
    r = np.arange(384)
    sm = np.arange(128)
    e = [(r[:, None] == 3 * sm[None, :] + a).astype(np.float32)
         for a in range(3)]
    bf = jnp.bfloat16
    return (jnp.asarray(bcat, bf), jnp.asarray(acat, bf),
            jnp.asarray(vpat, jnp.float32),
            t3j.astype(bf), jnp.asarray(e[0], bf),
            jnp.asarray(e[1], bf), jnp.asarray(e[2], bf))


def kernel(key_int_tensor, table):
    bcat, acat, vpat, t3, e0, e1, e2 = _tc_consts(table)
    idx2d = key_int_tensor.reshape(B // 128, 128)
    out = _tc_expand(idx2d, bcat, acat, vpat, t3, e0, e1, e2, NBLK, 0)
    return out.reshape(B_ROWS, B_COLS, D)


# speedup vs baseline: 1.4400x; 1.4400x over previous
import functools

import jax
import jax.numpy as jnp
from jax import lax
from jax.experimental import pallas as pl
from jax.experimental.pallas import tpu as pltpu
from jax.experimental.pallas import tpu_sc as plsc

B_ROWS = 16384
B_COLS = 200
D = 24
TPAD = 32
B = B_ROWS * B_COLS
NC, NS = 2, 16
NW = NC * NS
ROWS_PER_STEP = 2048
SUB = 4
SUBROWS = ROWS_PER_STEP // SUB
SUBGROUPS = SUBROWS // 16
SUBOUT = SUBROWS * D
OUT_PER_STEP = ROWS_PER_STEP * D
B_PER_W = B // NW
STEPS = B_PER_W // ROWS_PER_STEP


def _sc_lookup(idx_flat, tflat):
    mesh = plsc.VectorSubcoreMesh(core_axis_name="c", subcore_axis_name="s")

    @functools.partial(
        pl.kernel,
        mesh=mesh,
        compiler_params=pltpu.CompilerParams(
            use_tc_tiling_on_sc=False, needs_layout_passes=False),
        out_type=jax.ShapeDtypeStruct((B * D,), jnp.float32),
        scratch_types=[
            pltpu.VMEM((D * TPAD,), jnp.float32),
            pltpu.VMEM((ROWS_PER_STEP,), jnp.int32),
            pltpu.VMEM((ROWS_PER_STEP,), jnp.int32),
            pltpu.VMEM((OUT_PER_STEP + 16,), jnp.float32),
            pltpu.VMEM((OUT_PER_STEP + 16,), jnp.float32),
            pltpu.SemaphoreType.DMA,
            pltpu.SemaphoreType.DMA,
            pltpu.SemaphoreType.DMA,
            pltpu.SemaphoreType.DMA,
        ],
    )
    def k(idx_hbm, tab_hbm, out_hbm, tab_v,
          idx_v0, idx_v1, out_v0, out_v1, si0, si1, so0, so1):
        wid = lax.axis_index("s") * NC + lax.axis_index("c")
        row0 = wid * B_PER_W
        pltpu.sync_copy(tab_hbm, tab_v)

        idx_bufs = (idx_v0, idx_v1)
        out_bufs = (out_v0, out_v1)
        si = (si0, si1)
        so = (so0, so1)

        def idx_slice(it):
            base = pl.multiple_of(row0 + it * ROWS_PER_STEP, 8)
            return idx_hbm.at[pl.ds(base, ROWS_PER_STEP)]

        def out_sub(it, q4):
            base = pl.multiple_of(
                (row0 + it * ROWS_PER_STEP) * D + q4 * SUBOUT, 8)
            return out_hbm.at[pl.ds(base, SUBOUT)]

        def stage_sub(ob, q4):
            return ob.at[pl.ds(q4 * SUBOUT, SUBOUT)]

        pltpu.async_copy(idx_slice(0), idx_v0, si0)
        pltpu.async_copy(idx_slice(1), idx_v1, si1)

        def outer(i, carry):
            for b in range(2):
                it = 2 * i + b
                ib, ob, sib, sob = idx_bufs[b], out_bufs[b], si[b], so[b]
                pltpu.make_async_copy(idx_slice(it), ib, sib).wait()

                @pl.when(i > 0)
                def _wait_out():
                    for q4 in range(SUB):
                        pltpu.make_async_copy(
                            stage_sub(ob, q4), out_sub(it - 2, q4), sob).wait()

                def group(g, c):
                    obase = g * (16 * D)
                    idxv = ib[pl.ds(g * 16, 16)]
                    for u in range(16):
                        a = idxv[u] * TPAD
                        v1 = tab_v[pl.ds(a, 16)]
                        v2 = tab_v[pl.ds(a + 8, 16)]
                        ob[pl.ds(obase + u * D, 16)] = v1
                        ob[pl.ds(obase + u * D + 8, 16)] = v2
                    return c

                for q4 in range(SUB):
                    lax.fori_loop(q4 * SUBGROUPS, (q4 + 1) * SUBGROUPS,
                                  group, 0)
                    pltpu.async_copy(stage_sub(ob, q4), out_sub(it, q4), sob)

                @pl.when(it + 2 < STEPS)
                def _next_idx():
                    pltpu.async_copy(idx_slice(it + 2), ib, sib)
            return carry

        lax.fori_loop(0, STEPS // 2, outer, 0)
        for q4 in range(SUB):
            pltpu.make_async_copy(
                stage_sub(out_v0, q4), out_sub(STEPS - 2, q4), so0).wait()
            pltpu.make_async_copy(
                stage_sub(out_v1, q4), out_sub(STEPS - 1, q4), so1).wait()

    return k(idx_flat, tflat)


def kernel(key_int_tensor, table):
    tpad = jnp.zeros((D, TPAD), jnp.float32).at[:, :D].set(table[:D, :])
    out = _sc_lookup(key_int_tensor.reshape(B), tpad.reshape(D * TPAD))
    return out.reshape(B_ROWS, B_COLS, D)


# speedup vs baseline: 1.5633x; 1.0856x over previous
import functools

import jax
import jax.numpy as jnp
from jax import lax
from jax.experimental import pallas as pl
from jax.experimental.pallas import tpu as pltpu
from jax.experimental.pallas import tpu_sc as plsc

B_ROWS = 16384
B_COLS = 200
D = 24
TPAD = 32
B = B_ROWS * B_COLS
NC, NS = 2, 16
NW = NC * NS
ROWS_PER_STEP = 2048
SUB = 4
SUBROWS = ROWS_PER_STEP // SUB
SUBGROUPS = SUBROWS // 16
SUBOUT = SUBROWS * D
OUT_PER_STEP = ROWS_PER_STEP * D
B_PER_W = B // NW
STEPS = B_PER_W // ROWS_PER_STEP


def _sc_lookup(idx_flat, tflat):
    mesh = plsc.VectorSubcoreMesh(core_axis_name="c", subcore_axis_name="s")

    @functools.partial(
        pl.kernel,
        mesh=mesh,
        compiler_params=pltpu.CompilerParams(
            use_tc_tiling_on_sc=False, needs_layout_passes=False),
        out_type=jax.ShapeDtypeStruct((B * D,), jnp.float32),
        scratch_types=[
            pltpu.VMEM((D * TPAD,), jnp.float32),
            pltpu.VMEM((ROWS_PER_STEP,), jnp.int32),
            pltpu.VMEM((ROWS_PER_STEP,), jnp.int32),
            pltpu.VMEM((OUT_PER_STEP + 16,), jnp.float32),
            pltpu.VMEM((OUT_PER_STEP + 16,), jnp.float32),
            pltpu.SemaphoreType.DMA,
            pltpu.SemaphoreType.DMA,
            pltpu.SemaphoreType.DMA,
            pltpu.SemaphoreType.DMA,
        ],
    )
    def k(idx_hbm, tab_hbm, out_hbm, tab_v,
          idx_v0, idx_v1, out_v0, out_v1, si0, si1, so0, so1):
        wid = lax.axis_index("s") * NC + lax.axis_index("c")
        row0 = wid * B_PER_W
        pltpu.sync_copy(tab_hbm, tab_v)

        idx_bufs = (idx_v0, idx_v1)
        out_bufs = (out_v0, out_v1)
        si = (si0, si1)
        so = (so0, so1)

        def idx_slice(it):
            base = pl.multiple_of(row0 + it * ROWS_PER_STEP, 8)
            return idx_hbm.at[pl.ds(base, ROWS_PER_STEP)]

        def out_sub(it, q4):
            base = pl.multiple_of(
                (row0 + it * ROWS_PER_STEP) * D + q4 * SUBOUT, 8)
            return out_hbm.at[pl.ds(base, SUBOUT)]

        def stage_sub(ob, q4):
            return ob.at[pl.ds(q4 * SUBOUT, SUBOUT)]

        pltpu.async_copy(idx_slice(0), idx_v0, si0)
        pltpu.async_copy(idx_slice(1), idx_v1, si1)

        def outer(i, carry):
            for b in range(2):
                it = 2 * i + b
                ib, ob, sib, sob = idx_bufs[b], out_bufs[b], si[b], so[b]
                pltpu.make_async_copy(idx_slice(it), ib, sib).wait()

                @pl.when(i > 0)
                def _wait_out():
                    for q4 in range(SUB):
                        pltpu.make_async_copy(
                            stage_sub(ob, q4), out_sub(it - 2, q4), sob).wait()

                def group(g, c):
                    obase = g * (16 * D)
                    idxv = ib[pl.ds(g * 16, 16)]
                    a0 = idxv[0] * TPAD
                    v1p = tab_v[pl.ds(a0, 16)]
                    v2p = tab_v[pl.ds(a0 + 8, 16)]
                    for u in range(1, 16):
                        a = idxv[u] * TPAD
                        v1n = tab_v[pl.ds(a, 16)]
                        v2n = tab_v[pl.ds(a + 8, 16)]
                        ob[pl.ds(obase + (u - 1) * D, 16)] = v1p
                        ob[pl.ds(obase + (u - 1) * D + 8, 16)] = v2p
                        v1p, v2p = v1n, v2n
                    ob[pl.ds(obase + 15 * D, 16)] = v1p
                    ob[pl.ds(obase + 15 * D + 8, 16)] = v2p
                    return c

                for q4 in range(SUB):
                    lax.fori_loop(q4 * SUBGROUPS, (q4 + 1) * SUBGROUPS,
                                  group, 0)
                    pltpu.async_copy(stage_sub(ob, q4), out_sub(it, q4), sob)

                @pl.when(it + 2 < STEPS)
                def _next_idx():
                    pltpu.async_copy(idx_slice(it + 2), ib, sib)
            return carry

        lax.fori_loop(0, STEPS // 2, outer, 0)
        for q4 in range(SUB):
            pltpu.make_async_copy(
                stage_sub(out_v0, q4), out_sub(STEPS - 2, q4), so0).wait()
            pltpu.make_async_copy(
                stage_sub(out_v1, q4), out_sub(STEPS - 1, q4), so1).wait()

    return k(idx_flat, tflat)


def kernel(key_int_tensor, table):
    tpad = jnp.zeros((D, TPAD), jnp.float32).at[:, :D].set(table[:D, :])
    out = _sc_lookup(key_int_tensor.reshape(B), tpad.reshape(D * TPAD))
    return out.reshape(B_ROWS, B_COLS, D)


# speedup vs baseline: 1.5763x; 1.0083x over previous
import functools

import jax
import jax.numpy as jnp
from jax import lax
from jax.experimental import pallas as pl
from jax.experimental.pallas import tpu as pltpu
from jax.experimental.pallas import tpu_sc as plsc

B_ROWS = 16384
B_COLS = 200
D = 24
TPAD = 32
B = B_ROWS * B_COLS
NC, NS = 2, 16
NW = NC * NS
ROWS_PER_STEP = 2048
SUB = 4
SUBROWS = ROWS_PER_STEP // SUB
SUBGROUPS = SUBROWS // 16
SUBOUT = SUBROWS * D
OUT_PER_STEP = ROWS_PER_STEP * D
B_PER_W = B // NW
STEPS = B_PER_W // ROWS_PER_STEP


def _sc_lookup(idx_flat, tflat):
    mesh = plsc.VectorSubcoreMesh(core_axis_name="c", subcore_axis_name="s")

    @functools.partial(
        pl.kernel,
        mesh=mesh,
        compiler_params=pltpu.CompilerParams(
            use_tc_tiling_on_sc=False, needs_layout_passes=False),
        out_type=jax.ShapeDtypeStruct((B * D,), jnp.float32),
        scratch_types=[
            pltpu.VMEM((D * TPAD,), jnp.float32),
            pltpu.VMEM((ROWS_PER_STEP,), jnp.int32),
            pltpu.VMEM((ROWS_PER_STEP,), jnp.int32),
            pltpu.VMEM((OUT_PER_STEP + 16,), jnp.float32),
            pltpu.VMEM((OUT_PER_STEP + 16,), jnp.float32),
            pltpu.SemaphoreType.DMA,
            pltpu.SemaphoreType.DMA,
            pltpu.SemaphoreType.DMA,
            pltpu.SemaphoreType.DMA,
        ],
    )
    def k(idx_hbm, tab_hbm, out_hbm, tab_v,
          idx_v0, idx_v1, out_v0, out_v1, si0, si1, so0, so1):
        wid = lax.axis_index("s") * NC + lax.axis_index("c")
        row0 = wid * B_PER_W
        pltpu.sync_copy(tab_hbm, tab_v)

        idx_bufs = (idx_v0, idx_v1)
        out_bufs = (out_v0, out_v1)
        si = (si0, si1)
        so = (so0, so1)

        def idx_slice(it):
            base = pl.multiple_of(row0 + it * ROWS_PER_STEP, 8)
            return idx_hbm.at[pl.ds(base, ROWS_PER_STEP)]

        def out_sub(it, q4):
            base = pl.multiple_of(
                (row0 + it * ROWS_PER_STEP) * D + q4 * SUBOUT, 8)
            return out_hbm.at[pl.ds(base, SUBOUT)]

        def stage_sub(ob, q4):
            return ob.at[pl.ds(q4 * SUBOUT, SUBOUT)]

        pltpu.async_copy(idx_slice(0), idx_v0, si0)
        pltpu.async_copy(idx_slice(1), idx_v1, si1)

        def outer(i, carry):
            for b in range(2):
                it = 2 * i + b
                ib, ob, sib, sob = idx_bufs[b], out_bufs[b], si[b], so[b]
                pltpu.make_async_copy(idx_slice(it), ib, sib).wait()

                @pl.when(i > 0)
                def _wait_out():
                    for q4 in range(SUB):
                        pltpu.make_async_copy(
                            stage_sub(ob, q4), out_sub(it - 2, q4), sob).wait()

                def group(g, c):
                    obase = g * (16 * D)
                    idxv = ib[pl.ds(g * 16, 16)]
                    depth = 2
                    pend = []
                    for u in range(depth):
                        a = idxv[u] * TPAD
                        pend.append((tab_v[pl.ds(a, 16)],
                                     tab_v[pl.ds(a + 8, 16)]))
                    for u in range(depth, 16):
                        a = idxv[u] * TPAD
                        v1n = tab_v[pl.ds(a, 16)]
                        v2n = tab_v[pl.ds(a + 8, 16)]
                        v1p, v2p = pend.pop(0)
                        o = obase + (u - depth) * D
                        ob[pl.ds(o, 16)] = v1p
                        ob[pl.ds(o + 8, 16)] = v2p
                        pend.append((v1n, v2n))
                    for w in range(depth):
                        v1p, v2p = pend.pop(0)
                        o = obase + (16 - depth + w) * D
                        ob[pl.ds(o, 16)] = v1p
                        ob[pl.ds(o + 8, 16)] = v2p
                    return c

                for q4 in range(SUB):
                    lax.fori_loop(q4 * SUBGROUPS, (q4 + 1) * SUBGROUPS,
                                  group, 0)
                    pltpu.async_copy(stage_sub(ob, q4), out_sub(it, q4), sob)

                @pl.when(it + 2 < STEPS)
                def _next_idx():
                    pltpu.async_copy(idx_slice(it + 2), ib, sib)
            return carry

        lax.fori_loop(0, STEPS // 2, outer, 0)
        for q4 in range(SUB):
            pltpu.make_async_copy(
                stage_sub(out_v0, q4), out_sub(STEPS - 2, q4), so0).wait()
            pltpu.make_async_copy(
                stage_sub(out_v1, q4), out_sub(STEPS - 1, q4), so1).wait()

    return k(idx_flat, tflat)


def kernel(key_int_tensor, table):
    tpad = jnp.zeros((D, TPAD), jnp.float32).at[:, :D].set(table[:D, :])
    out = _sc_lookup(key_int_tensor.reshape(B), tpad.reshape(D * TPAD))
    return out.reshape(B_ROWS, B_COLS, D)
